# baseline (speedup 1.0000x reference)
import functools

import jax
import jax.numpy as jnp
from jax import lax
from jax.experimental import pallas as pl
from jax.experimental.pallas import tpu as pltpu
from jax.experimental.pallas import tpu_sc as plsc

NC = 2
NS = 16
CHUNK = 128
SUP = 4
CW = 128


def _ceil_to(x, m):
  return (x + m - 1) // m * m


NBUF = 3


def _edge_pipeline(tab, sidx, didx, rows, gsems, ssems, acc):
  gd = [None] * NBUF
  sd = [None] * NBUF
  gd[0] = pltpu.async_copy(tab.at[sidx.at[0]], rows[0], gsems[0])
  for j in range(SUP):
    b = j % NBUF
    gd[b].wait()
    if j + 1 < SUP:
      nb = (j + 1) % NBUF
      if sd[nb] is not None:
        sd[nb].wait()
      gd[nb] = pltpu.async_copy(tab.at[sidx.at[j + 1]], rows[nb], gsems[nb])
    sd[b] = pltpu.async_copy(rows[b], acc.at[didx.at[j]], ssems[b], add=True)
  for b in range(NBUF):
    if sd[b] is not None:
      sd[b].wait()


def _make_agg1(n_pad, n_sup_per_tile, d):
  rows_per_tile = n_pad // NS

  mesh = plsc.VectorSubcoreMesh(core_axis_name="c", subcore_axis_name="s")

  @functools.partial(
      pl.kernel,
      mesh=mesh,
      out_type=jax.ShapeDtypeStruct((NC, n_pad, d), jnp.float32),
      scratch_types=[
          pltpu.VMEM((SUP, CHUNK), jnp.int32),
          pltpu.VMEM((SUP, CHUNK), jnp.int32),
          [pltpu.VMEM((CHUNK, d), jnp.float32)] * NBUF,
          [pltpu.SemaphoreType.DMA] * NBUF,
          [pltpu.SemaphoreType.DMA] * NBUF,
          pltpu.VMEM_SHARED((n_pad, d), jnp.float32),
      ],
  )
  def agg1(x_hbm, src_hbm, dst_hbm, zero_hbm, s_out, sidx, didx, rows,
           gsems, ssems, acc):
    c = lax.axis_index("c")
    t = lax.axis_index("s")

    r0 = t * rows_per_tile
    pltpu.sync_copy(zero_hbm, acc.at[pl.ds(r0, rows_per_tile)])
    plsc.subcore_barrier()

    chunk0 = (c * NS + t) * n_sup_per_tile * SUP

    def outer(sc, _):
      pltpu.sync_copy(src_hbm.at[pl.ds(chunk0 + sc * SUP, SUP)], sidx)
      pltpu.sync_copy(dst_hbm.at[pl.ds(chunk0 + sc * SUP, SUP)], didx)
      _edge_pipeline(x_hbm, sidx, didx, rows, gsems, ssems, acc)
      return 0

    lax.fori_loop(0, n_sup_per_tile, outer, 0)
    plsc.subcore_barrier()

    pltpu.sync_copy(acc.at[pl.ds(r0, rows_per_tile)],
                    s_out.at[c].at[pl.ds(r0, rows_per_tile)])

  return agg1


def _make_cnt(n_pad, n_sup_per_tile):
  rows_per_tile = n_pad // NS

  mesh = plsc.VectorSubcoreMesh(core_axis_name="c", subcore_axis_name="s")

  @functools.partial(
      pl.kernel,
      mesh=mesh,
      out_type=jax.ShapeDtypeStruct((NC, n_pad, CW), jnp.float32),
      scratch_types=[
          pltpu.VMEM((SUP, CHUNK), jnp.int32),
          pltpu.VMEM((CHUNK, CW), jnp.float32),
          pltpu.SemaphoreType.DMA,
          pltpu.VMEM_SHARED((n_pad, CW), jnp.float32),
      ],
  )
  def cntk(dst_hbm, zero_hbm, ones_hbm, c_out, didx, ones, ssem, cnt):
    c = lax.axis_index("c")
    t = lax.axis_index("s")

    pltpu.sync_copy(ones_hbm, ones)
    r0 = t * rows_per_tile
    pltpu.sync_copy(zero_hbm, cnt.at[pl.ds(r0, rows_per_tile)])
    plsc.subcore_barrier()

    chunk0 = (c * NS + t) * n_sup_per_tile * SUP

    def outer(sc, _):
      pltpu.sync_copy(dst_hbm.at[pl.ds(chunk0 + sc * SUP, SUP)], didx)
      descs = [
          pltpu.async_copy(ones, cnt.at[didx.at[j]], ssem, add=True)
          for j in range(SUP)
      ]
      for dsc in descs:
        dsc.wait()
      return 0

    lax.fori_loop(0, n_sup_per_tile, outer, 0)
    plsc.subcore_barrier()

    pltpu.sync_copy(cnt.at[pl.ds(r0, rows_per_tile)],
                    c_out.at[c].at[pl.ds(r0, rows_per_tile)])

  return cntk


def _make_agg2(n_pad, n_sup_per_tile, d):
  rows_per_tile = n_pad // NS

  mesh = plsc.VectorSubcoreMesh(core_axis_name="c", subcore_axis_name="s")

  @functools.partial(
      pl.kernel,
      mesh=mesh,
      out_type=jax.ShapeDtypeStruct((NC, n_pad, d), jnp.float32),
      scratch_types=[
          pltpu.VMEM((SUP, CHUNK), jnp.int32),
          pltpu.VMEM((SUP, CHUNK), jnp.int32),
          [pltpu.VMEM((CHUNK, d), jnp.float32)] * NBUF,
          [pltpu.SemaphoreType.DMA] * NBUF,
          [pltpu.SemaphoreType.DMA] * NBUF,
          pltpu.VMEM_SHARED((n_pad, d), jnp.float32),
      ],
  )
  def agg2(h1a_hbm, h1b_hbm, src_hbm, dst_hbm, zero_hbm, s_out, sidx, didx,
           rows, gsems, ssems, acc):
    c = lax.axis_index("c")
    t = lax.axis_index("s")

    r0 = t * rows_per_tile
    pltpu.sync_copy(zero_hbm, acc.at[pl.ds(r0, rows_per_tile)])
    plsc.subcore_barrier()

    chunk0 = t * n_sup_per_tile * SUP

    def make_outer(tab):
      def outer(sc, _):
        pltpu.sync_copy(src_hbm.at[pl.ds(chunk0 + sc * SUP, SUP)], sidx)
        pltpu.sync_copy(dst_hbm.at[pl.ds(chunk0 + sc * SUP, SUP)], didx)
        _edge_pipeline(tab, sidx, didx, rows, gsems, ssems, acc)
        return 0

      return outer

    @pl.when(c == 0)
    def _():
      lax.fori_loop(0, n_sup_per_tile, make_outer(h1a_hbm), 0)

    @pl.when(c == 1)
    def _():
      lax.fori_loop(0, n_sup_per_tile, make_outer(h1b_hbm), 0)

    plsc.subcore_barrier()
    pltpu.sync_copy(acc.at[pl.ds(r0, rows_per_tile)],
                    s_out.at[c].at[pl.ds(r0, rows_per_tile)])

  return agg2


def _l1_body(s0, s1, c0, c1, x, wl1t, bl1, wr1t, h1a, h1b):
  cnt = jnp.maximum(c0[:, :1] + c1[:, :1], 1.0)
  mean = (s0[...] + s1[...]) / cnt
  h = (jnp.dot(mean, wl1t[...], preferred_element_type=jnp.float32)
       + jnp.dot(x[...], wr1t[...], preferred_element_type=jnp.float32)
       + bl1[...])
  h = jnp.maximum(h, 0.0)
  h1a[...] = h[:, :128]
  h1b[...] = h[:, 128:]


def _l2_body(s2a, s2b, c0, c1, h1a, h1b, bcol, wl2ta, wl2tb, wr2ta, wr2tb,
             bl2, wlint, blin, out, pooled, cntc):
  i = pl.program_id(0)

  @pl.when(i == 0)
  def _():
    pooled[...] = jnp.zeros_like(pooled)
    cntc[...] = jnp.zeros_like(cntc)

  cnt = jnp.maximum(c0[:, :1] + c1[:, :1], 1.0)
  h = (jnp.dot(s2a[...] / cnt, wl2ta[...], preferred_element_type=jnp.float32)
       + jnp.dot(s2b[...] / cnt, wl2tb[...],
                 preferred_element_type=jnp.float32)
       + jnp.dot(h1a[...], wr2ta[...], preferred_element_type=jnp.float32)
       + jnp.dot(h1b[...], wr2tb[...], preferred_element_type=jnp.float32)
       + bl2[...])
  h = jnp.maximum(h, 0.0)

  b = bcol[0]
  gids = lax.broadcasted_iota(jnp.int32, (b.shape[0], 128), 1).astype(
      jnp.float32)
  oh = (b == gids).astype(jnp.float32)
  pooled[...] += lax.dot_general(oh, h, (((0,), (0,)), ((), ())),
                                 preferred_element_type=jnp.float32)
  cntc[...] += lax.dot_general(oh, jnp.ones((b.shape[0], 8), jnp.float32),
                               (((0,), (0,)), ((), ())),
                               preferred_element_type=jnp.float32)

  @pl.when(i == pl.num_programs(0) - 1)
  def _():
    cg = jnp.maximum(cntc[:, :1], 1.0)
    out[...] = (jnp.dot(pooled[...] / cg, wlint[...],
                        preferred_element_type=jnp.float32) + blin[...])


def kernel(x, edge_index, batch, Wl1, bl1, Wr1, Wl2, bl2, Wr2, Wlin, blin):
  n, d = x.shape
  e = edge_index.shape[1]
  h_dim = Wl1.shape[0]
  g = 128

  e_pad = _ceil_to(e, CHUNK * SUP * NS * NC)
  n_pad = _ceil_to(n + 1, NS * 8)
  src = jnp.concatenate(
      [edge_index[0], jnp.zeros((e_pad - e,), jnp.int32)]).reshape(-1, CHUNK)
  dst = jnp.concatenate(
      [edge_index[1], jnp.full((e_pad - e,), n, jnp.int32)]).reshape(-1, CHUNK)

  nsup1 = e_pad // (CHUNK * SUP * NS * NC)
  nsup2 = e_pad // (CHUNK * SUP * NS)

  zero_blk = jnp.zeros((n_pad // NS, 128), jnp.float32)
  ones_blk = jnp.ones((CHUNK, CW), jnp.float32)

  s1 = _make_agg1(n_pad, nsup1, d)(x, src, dst, zero_blk)
  c1 = _make_cnt(n_pad, nsup1)(dst, zero_blk, ones_blk)

  nb = 5
  blk = n // nb
  spec = lambda r, c_: pl.BlockSpec((blk, c_), lambda i: (i, 0))
  wspec = lambda r, c_: pl.BlockSpec((r, c_), lambda i: (0, 0))
  h1a, h1b = pl.pallas_call(
      _l1_body,
      grid=(nb,),
      in_specs=[
          spec(blk, d), spec(blk, d),
          spec(blk, CW), spec(blk, CW),
          spec(blk, d),
          wspec(d, h_dim), wspec(1, h_dim), wspec(d, h_dim),
      ],
      out_specs=[spec(blk, 128), spec(blk, 128)],
      out_shape=[jax.ShapeDtypeStruct((n, 128), jnp.float32)] * 2,
  )(s1[0, :n], s1[1, :n], c1[0, :n], c1[1, :n], x,
    Wl1.T, bl1.reshape(1, -1), Wr1.T)

  s2 = _make_agg2(n_pad, nsup2, 128)(h1a, h1b, src, dst, zero_blk)

  bcol = batch.astype(jnp.float32).reshape(nb, blk, 1)
  out = pl.pallas_call(
      _l2_body,
      grid=(nb,),
      in_specs=[
          spec(blk, 128), spec(blk, 128),
          spec(blk, CW), spec(blk, CW),
          spec(blk, 128), spec(blk, 128),
          pl.BlockSpec((1, blk, 1), lambda i: (i, 0, 0)),
          wspec(128, h_dim), wspec(128, h_dim),
          wspec(128, h_dim), wspec(128, h_dim),
          wspec(1, h_dim),
          wspec(h_dim, h_dim), wspec(1, h_dim),
      ],
      out_specs=pl.BlockSpec((g, h_dim), lambda i: (0, 0)),
      out_shape=jax.ShapeDtypeStruct((g, h_dim), jnp.float32),
      scratch_shapes=[
          pltpu.VMEM((g, h_dim), jnp.float32),
          pltpu.VMEM((g, 8), jnp.float32),
      ],
      compiler_params=pltpu.CompilerParams(
          dimension_semantics=("arbitrary",)),
  )(s2[0, :n], s2[1, :n], c1[0, :n], c1[1, :n], h1a, h1b, bcol,
    Wl2[:, :128].T, Wl2[:, 128:].T, Wr2[:, :128].T, Wr2[:, 128:].T,
    bl2.reshape(1, -1), Wlin.T, blin.reshape(1, -1))
  return out


# speedup vs baseline: 1.0503x; 1.0503x over previous
import functools

import jax
import jax.numpy as jnp
from jax import lax
from jax.experimental import pallas as pl
from jax.experimental.pallas import tpu as pltpu
from jax.experimental.pallas import tpu_sc as plsc

NC = 2
NS = 16
CHUNK = 128
SUP = 16
CW = 128


def _ceil_to(x, m):
  return (x + m - 1) // m * m


NBUF = 2


def _edge_pipeline(tab, sidx, didx, rows, gsems, ssems, acc):
  gd = [None] * NBUF
  sd = [None] * NBUF
  gd[0] = pltpu.async_copy(tab.at[sidx.at[0]], rows[0], gsems[0])
  for j in range(SUP):
    b = j % NBUF
    gd[b].wait()
    if j + 1 < SUP:
      nb = (j + 1) % NBUF
      if sd[nb] is not None:
        sd[nb].wait()
      gd[nb] = pltpu.async_copy(tab.at[sidx.at[j + 1]], rows[nb], gsems[nb])
    sd[b] = pltpu.async_copy(rows[b], acc.at[didx.at[j]], ssems[b], add=True)
  for b in range(NBUF):
    if sd[b] is not None:
      sd[b].wait()


def _make_agg1(n_pad, n_sup_per_tile, d):
  rows_per_tile = n_pad // NS

  mesh = plsc.VectorSubcoreMesh(core_axis_name="c", subcore_axis_name="s")

  @functools.partial(
      pl.kernel,
      mesh=mesh,
      out_type=jax.ShapeDtypeStruct((NC, n_pad, d), jnp.float32),
      scratch_types=[
          pltpu.VMEM((SUP, CHUNK), jnp.int32),
          pltpu.VMEM((SUP, CHUNK), jnp.int32),
          [pltpu.VMEM((CHUNK, d), jnp.float32)] * NBUF,
          [pltpu.SemaphoreType.DMA] * NBUF,
          [pltpu.SemaphoreType.DMA] * NBUF,
          pltpu.VMEM_SHARED((n_pad, d), jnp.float32),
      ],
  )
  def agg1(x0_hbm, x1_hbm, src_hbm, dst_hbm, zero_hbm, s_out, sidx, didx,
           rows, gsems, ssems, acc):
    c = lax.axis_index("c")
    t = lax.axis_index("s")

    r0 = t * rows_per_tile
    pltpu.sync_copy(zero_hbm, acc.at[pl.ds(r0, rows_per_tile)])
    plsc.subcore_barrier()

    chunk0 = (c * NS + t) * n_sup_per_tile * SUP

    def make_outer(tab):
      def outer(sc, _):
        pltpu.sync_copy(src_hbm.at[pl.ds(chunk0 + sc * SUP, SUP)], sidx)
        pltpu.sync_copy(dst_hbm.at[pl.ds(chunk0 + sc * SUP, SUP)], didx)
        _edge_pipeline(tab, sidx, didx, rows, gsems, ssems, acc)
        return 0

      return outer

    @pl.when(c == 0)
    def _():
      lax.fori_loop(0, n_sup_per_tile, make_outer(x0_hbm), 0)

    @pl.when(c == 1)
    def _():
      lax.fori_loop(0, n_sup_per_tile, make_outer(x1_hbm), 0)
    plsc.subcore_barrier()

    pltpu.sync_copy(acc.at[pl.ds(r0, rows_per_tile)],
                    s_out.at[c].at[pl.ds(r0, rows_per_tile)])

  return agg1


def _make_cnt(n_pad, n_sup_per_tile):
  rows_per_tile = n_pad // NS

  mesh = plsc.VectorSubcoreMesh(core_axis_name="c", subcore_axis_name="s")

  @functools.partial(
      pl.kernel,
      mesh=mesh,
      out_type=jax.ShapeDtypeStruct((NC, n_pad, CW), jnp.float32),
      scratch_types=[
          pltpu.VMEM((SUP, CHUNK), jnp.int32),
          pltpu.VMEM((CHUNK, CW), jnp.float32),
          pltpu.SemaphoreType.DMA,
          pltpu.VMEM_SHARED((n_pad, CW), jnp.float32),
      ],
  )
  def cntk(dst_hbm, zero_hbm, ones_hbm, c_out, didx, ones, ssem, cnt):
    c = lax.axis_index("c")
    t = lax.axis_index("s")

    pltpu.sync_copy(ones_hbm, ones)
    r0 = t * rows_per_tile
    pltpu.sync_copy(zero_hbm, cnt.at[pl.ds(r0, rows_per_tile)])
    plsc.subcore_barrier()

    chunk0 = (c * NS + t) * n_sup_per_tile * SUP

    def outer(sc, _):
      pltpu.sync_copy(dst_hbm.at[pl.ds(chunk0 + sc * SUP, SUP)], didx)
      descs = [
          pltpu.async_copy(ones, cnt.at[didx.at[j]], ssem, add=True)
          for j in range(SUP)
      ]
      for dsc in descs:
        dsc.wait()
      return 0

    lax.fori_loop(0, n_sup_per_tile, outer, 0)
    plsc.subcore_barrier()

    pltpu.sync_copy(cnt.at[pl.ds(r0, rows_per_tile)],
                    c_out.at[c].at[pl.ds(r0, rows_per_tile)])

  return cntk


def _make_agg2(n_pad, n_sup_per_tile, d):
  rows_per_tile = n_pad // NS

  mesh = plsc.VectorSubcoreMesh(core_axis_name="c", subcore_axis_name="s")

  @functools.partial(
      pl.kernel,
      mesh=mesh,
      out_type=jax.ShapeDtypeStruct((NC, n_pad, d), jnp.float32),
      scratch_types=[
          pltpu.VMEM((SUP, CHUNK), jnp.int32),
          pltpu.VMEM((SUP, CHUNK), jnp.int32),
          [pltpu.VMEM((CHUNK, d), jnp.float32)] * NBUF,
          [pltpu.SemaphoreType.DMA] * NBUF,
          [pltpu.SemaphoreType.DMA] * NBUF,
          pltpu.VMEM_SHARED((n_pad, d), jnp.float32),
      ],
  )
  def agg2(h1a_hbm, h1b_hbm, src_hbm, dst_hbm, zero_hbm, s_out, sidx, didx,
           rows, gsems, ssems, acc):
    c = lax.axis_index("c")
    t = lax.axis_index("s")

    r0 = t * rows_per_tile
    pltpu.sync_copy(zero_hbm, acc.at[pl.ds(r0, rows_per_tile)])
    plsc.subcore_barrier()

    chunk0 = t * n_sup_per_tile * SUP

    def make_outer(tab):
      def outer(sc, _):
        pltpu.sync_copy(src_hbm.at[pl.ds(chunk0 + sc * SUP, SUP)], sidx)
        pltpu.sync_copy(dst_hbm.at[pl.ds(chunk0 + sc * SUP, SUP)], didx)
        _edge_pipeline(tab, sidx, didx, rows, gsems, ssems, acc)
        return 0

      return outer

    @pl.when(c == 0)
    def _():
      lax.fori_loop(0, n_sup_per_tile, make_outer(h1a_hbm), 0)

    @pl.when(c == 1)
    def _():
      lax.fori_loop(0, n_sup_per_tile, make_outer(h1b_hbm), 0)

    plsc.subcore_barrier()
    pltpu.sync_copy(acc.at[pl.ds(r0, rows_per_tile)],
                    s_out.at[c].at[pl.ds(r0, rows_per_tile)])

  return agg2


def _l1_body(s0, s1, c0, c1, x, wl1t, bl1, wr1t, h1a, h1b):
  cnt = jnp.maximum(c0[:, :1] + c1[:, :1], 1.0)
  mean = (s0[...] + s1[...]) / cnt
  h = (jnp.dot(mean, wl1t[...], preferred_element_type=jnp.float32)
       + jnp.dot(x[...], wr1t[...], preferred_element_type=jnp.float32)
       + bl1[...])
  h = jnp.maximum(h, 0.0)
  h1a[...] = h[:, :128]
  h1b[...] = h[:, 128:]


def _l2_body(s2a, s2b, c0, c1, h1a, h1b, bcol, wl2ta, wl2tb, wr2ta, wr2tb,
             bl2, wlint, blin, out, pooled, cntc):
  i = pl.program_id(0)

  @pl.when(i == 0)
  def _():
    pooled[...] = jnp.zeros_like(pooled)
    cntc[...] = jnp.zeros_like(cntc)

  cnt = jnp.maximum(c0[:, :1] + c1[:, :1], 1.0)
  h = (jnp.dot(s2a[...] / cnt, wl2ta[...], preferred_element_type=jnp.float32)
       + jnp.dot(s2b[...] / cnt, wl2tb[...],
                 preferred_element_type=jnp.float32)
       + jnp.dot(h1a[...], wr2ta[...], preferred_element_type=jnp.float32)
       + jnp.dot(h1b[...], wr2tb[...], preferred_element_type=jnp.float32)
       + bl2[...])
  h = jnp.maximum(h, 0.0)

  b = bcol[0]
  gids = lax.broadcasted_iota(jnp.int32, (b.shape[0], 128), 1).astype(
      jnp.float32)
  oh = (b == gids).astype(jnp.float32)
  pooled[...] += lax.dot_general(oh, h, (((0,), (0,)), ((), ())),
                                 preferred_element_type=jnp.float32)
  cntc[...] += lax.dot_general(oh, jnp.ones((b.shape[0], 8), jnp.float32),
                               (((0,), (0,)), ((), ())),
                               preferred_element_type=jnp.float32)

  @pl.when(i == pl.num_programs(0) - 1)
  def _():
    cg = jnp.maximum(cntc[:, :1], 1.0)
    out[...] = (jnp.dot(pooled[...] / cg, wlint[...],
                        preferred_element_type=jnp.float32) + blin[...])


def kernel(x, edge_index, batch, Wl1, bl1, Wr1, Wl2, bl2, Wr2, Wlin, blin):
  n, d = x.shape
  e = edge_index.shape[1]
  h_dim = Wl1.shape[0]
  g = 128

  e_pad = _ceil_to(e, CHUNK * SUP * NS * NC)
  n_pad = _ceil_to(n + 1, NS * 8)
  src = jnp.concatenate(
      [edge_index[0], jnp.zeros((e_pad - e,), jnp.int32)]).reshape(-1, CHUNK)
  dst = jnp.concatenate(
      [edge_index[1], jnp.full((e_pad - e,), n, jnp.int32)]).reshape(-1, CHUNK)

  nsup1 = e_pad // (CHUNK * SUP * NS * NC)
  nsup2 = e_pad // (CHUNK * SUP * NS)

  zero_blk = jnp.zeros((n_pad // NS, 128), jnp.float32)
  ones_blk = jnp.ones((CHUNK, CW), jnp.float32)

  x_dup = x + jnp.zeros((1, 1), x.dtype)
  s1 = _make_agg1(n_pad, nsup1, d)(x, x_dup, src, dst, zero_blk)
  c1 = _make_cnt(n_pad, nsup1)(dst, zero_blk, ones_blk)

  nb = 5
  blk = n // nb
  spec = lambda r, c_: pl.BlockSpec((blk, c_), lambda i: (i, 0))
  wspec = lambda r, c_: pl.BlockSpec((r, c_), lambda i: (0, 0))
  h1a, h1b = pl.pallas_call(
      _l1_body,
      grid=(nb,),
      in_specs=[
          spec(blk, d), spec(blk, d),
          spec(blk, CW), spec(blk, CW),
          spec(blk, d),
          wspec(d, h_dim), wspec(1, h_dim), wspec(d, h_dim),
      ],
      out_specs=[spec(blk, 128), spec(blk, 128)],
      out_shape=[jax.ShapeDtypeStruct((n, 128), jnp.float32)] * 2,
  )(s1[0, :n], s1[1, :n], c1[0, :n], c1[1, :n], x,
    Wl1.T, bl1.reshape(1, -1), Wr1.T)

  s2 = _make_agg2(n_pad, nsup2, 128)(h1a, h1b, src, dst, zero_blk)

  bcol = batch.astype(jnp.float32).reshape(nb, blk, 1)
  out = pl.pallas_call(
      _l2_body,
      grid=(nb,),
      in_specs=[
          spec(blk, 128), spec(blk, 128),
          spec(blk, CW), spec(blk, CW),
          spec(blk, 128), spec(blk, 128),
          pl.BlockSpec((1, blk, 1), lambda i: (i, 0, 0)),
          wspec(128, h_dim), wspec(128, h_dim),
          wspec(128, h_dim), wspec(128, h_dim),
          wspec(1, h_dim),
          wspec(h_dim, h_dim), wspec(1, h_dim),
      ],
      out_specs=pl.BlockSpec((g, h_dim), lambda i: (0, 0)),
      out_shape=jax.ShapeDtypeStruct((g, h_dim), jnp.float32),
      scratch_shapes=[
          pltpu.VMEM((g, h_dim), jnp.float32),
          pltpu.VMEM((g, 8), jnp.float32),
      ],
      compiler_params=pltpu.CompilerParams(
          dimension_semantics=("arbitrary",)),
  )(s2[0, :n], s2[1, :n], c1[0, :n], c1[1, :n], h1a, h1b, bcol,
    Wl2[:, :128].T, Wl2[:, 128:].T, Wr2[:, :128].T, Wr2[:, 128:].T,
    bl2.reshape(1, -1), Wlin.T, blin.reshape(1, -1))
  return out
